# Initial kernel scaffold; baseline (speedup 1.0000x reference)
import jax
import jax.numpy as jnp
from jax.experimental import pallas as pl


def kernel(x, edge_index, Wl0, bl0, Wr0, g0, be0, Wl1, bl1, Wr1, g1, be1, Wl2, bl2, Wr2, g2, be2, Wc1, bc1, Wc2, bc2):
    raise NotImplementedError("write your pallas kernel here")


# speedup vs baseline: 3.2055x; 3.2055x over previous
import functools

import jax
import jax.numpy as jnp
from jax import lax
from jax.experimental import pallas as pl
from jax.experimental.pallas import tpu as pltpu
from jax.experimental.pallas import tpu_sc as plsc

N = 10000
E = 320000
NC = 2
NS = 16
NW = NC * NS
EPW = E // NW
K = 80
NCHUNK = EPW // K
NP = 10240
RPT = NP // NS
F = 128
CW = 16
ZR = 16


def _make_agg(P, with_count):
  mesh = plsc.VectorSubcoreMesh(core_axis_name="c", subcore_axis_name="s",
                                num_cores=NC, num_subcores=NS)
  out_type = [jax.ShapeDtypeStruct((NC, P, NP, F), jnp.float32)]
  if with_count:
    out_type.append(jax.ShapeDtypeStruct((NC, NP, F), jnp.float32))
  scratch = [
      pltpu.VMEM((1, K), jnp.int32),
      pltpu.VMEM((1, K), jnp.int32),
      pltpu.VMEM((K, F), jnp.float32),
      pltpu.VMEM((ZR, F), jnp.float32),
      pltpu.VMEM_SHARED((NP, F), jnp.float32),
      pltpu.SemaphoreType.DMA,
  ]

  def body(*refs):
    parts = refs[:P]
    srcr, dstr = refs[P], refs[P + 1]
    out = refs[P + 2]
    i = P + 3
    if with_count:
      outc = refs[i]
      i += 1
    srcv, dstv, rows, zb, acc, sem = refs[i:i + 6]

    c = lax.axis_index("c")
    s = lax.axis_index("s")
    wid = c * NS + s

    zeros16 = jnp.zeros((16,), jnp.float32)

    def zrow(r, carry):
      for t in range(F // 16):
        zb[r, pl.ds(t * 16, 16)] = zeros16
      return carry

    lax.fori_loop(0, ZR, zrow, 0)

    def zstripe(q, carry):
      pltpu.sync_copy(zb, acc.at[pl.ds(s * RPT + q * ZR, ZR)])
      return carry

    for p in range(P):
      lax.fori_loop(0, RPT // ZR, zstripe, 0)
      plsc.subcore_barrier()

      def chunk(j, carry):
        pltpu.sync_copy(srcr.at[wid, j], srcv)
        pltpu.sync_copy(dstr.at[wid, j], dstv)
        pltpu.async_copy(parts[p].at[srcv.at[0]], rows, sem).wait()
        pltpu.sync_copy(rows, acc.at[dstv.at[0]], add=True)
        return carry

      lax.fori_loop(0, NCHUNK, chunk, 0)
      plsc.subcore_barrier()
      pltpu.sync_copy(acc.at[pl.ds(s * RPT, RPT)],
                      out.at[c, p, pl.ds(s * RPT, RPT)])

    if with_count:
      ones16 = jnp.ones((16,), jnp.float32)

      def orow(r, carry):
        for t in range(F // 16):
          rows[r, pl.ds(t * 16, 16)] = ones16
        return carry

      lax.fori_loop(0, K, orow, 0)
      lax.fori_loop(0, RPT // ZR, zstripe, 0)
      plsc.subcore_barrier()

      def cchunk(j, carry):
        pltpu.sync_copy(dstr.at[wid, j], dstv)
        pltpu.sync_copy(rows, acc.at[dstv.at[0]], add=True)
        return carry

      lax.fori_loop(0, NCHUNK, cchunk, 0)
      plsc.subcore_barrier()
      pltpu.sync_copy(acc.at[pl.ds(s * RPT, RPT)],
                      outc.at[c, pl.ds(s * RPT, RPT)])

  return pl.kernel(body, out_type=out_type, mesh=mesh, scratch_types=scratch)


@functools.lru_cache(maxsize=None)
def _agg(P, with_count):
  return _make_agg(P, with_count)


def _tc_mean_body(P, s_ref, cnt_ref, out_ref):
  cnt = cnt_ref[0, :N, 0:1] + cnt_ref[1, :N, 0:1]
  inv = 1.0 / jnp.maximum(cnt, 1.0)
  parts = [s_ref[0, p, :N] + s_ref[1, p, :N] for p in range(P)]
  mean = jnp.concatenate(parts, axis=1) if P > 1 else parts[0]
  out_ref[...] = mean * inv


def _tc_sage_body(mean_ref, h_ref, wl, bl, wr, g, be, out_ref):
  z = (jnp.dot(mean_ref[...], wl[...], preferred_element_type=jnp.float32)
       + bl[...][None, :]
       + jnp.dot(h_ref[...], wr[...], preferred_element_type=jnp.float32))
  m = jnp.mean(z, axis=0)
  v = jnp.mean((z - m[None, :]) ** 2, axis=0)
  hn = (z - m[None, :]) * lax.rsqrt(v + 1e-5) * g[...][None, :] + be[...][None, :]
  out_ref[...] = jnp.maximum(hn, 0.0)


def _tc_head_body(h_ref, wc1, bc1, wc2, bc2, out_ref):
  c1 = jnp.maximum(
      jnp.dot(h_ref[...], wc1[...], preferred_element_type=jnp.float32)
      + bc1[...][None, :], 0.0)
  out_ref[...] = (jnp.dot(c1, wc2[...], preferred_element_type=jnp.float32)
                  + bc2[...][None, :])


def _tc_mean(P):
  return pl.pallas_call(
      functools.partial(_tc_mean_body, P),
      out_shape=jax.ShapeDtypeStruct((N, P * F), jnp.float32))


_tc_sage = pl.pallas_call(
    _tc_sage_body, out_shape=jax.ShapeDtypeStruct((N, 256), jnp.float32))
_tc_head = pl.pallas_call(
    _tc_head_body, out_shape=jax.ShapeDtypeStruct((N, 2), jnp.float32))


def kernel(x, edge_index, Wl0, bl0, Wr0, g0, be0, Wl1, bl1, Wr1, g1, be1,
           Wl2, bl2, Wr2, g2, be2, Wc1, bc1, Wc2, bc2):
  src = edge_index[0].reshape(NW, NCHUNK, 1, K)
  dst = edge_index[1].reshape(NW, NCHUNK, 1, K)

  s0, cnt = _agg(1, True)(x, src, dst)
  h1 = _tc_sage(_tc_mean(1)(s0, cnt), x, Wl0.T, bl0, Wr0.T, g0, be0)

  s1, = _agg(2, False)(h1[:, :F], h1[:, F:], src, dst)
  h2 = _tc_sage(_tc_mean(2)(s1, cnt), h1, Wl1.T, bl1, Wr1.T, g1, be1)

  s2, = _agg(2, False)(h2[:, :F], h2[:, F:], src, dst)
  h3 = _tc_sage(_tc_mean(2)(s2, cnt), h2, Wl2.T, bl2, Wr2.T, g2, be2)
  return _tc_head(h3, Wc1.T, bc1, Wc2.T, bc2)
